# Initial kernel scaffold; baseline (speedup 1.0000x reference)
import jax
import jax.numpy as jnp
from jax.experimental import pallas as pl


def kernel(x, edge_attr, edge_index, W_enc, W_m1, W_m2, W_u, W_ih, W_hh, W_mst, W_p1, W_p2):
    raise NotImplementedError("write your pallas kernel here")


# speedup vs baseline: 1.1980x; 1.1980x over previous
import jax
import jax.numpy as jnp
from jax.experimental import pallas as pl

G = 16


def _leaky(v):
    return jnp.where(v >= 0, v, 0.01 * v)


def kernel(x, edge_attr, edge_index, W_enc, W_m1, W_m2, W_u, W_ih, W_hh, W_mst, W_p1, W_p2):
    n = x.shape[0]
    L = W_m2.shape[0]
    src = edge_index[0]
    dst = edge_index[1]
    steps = x.shape[1]
    ea = edge_attr

    W_enc0 = W_enc[0]
    W_enc1 = W_enc[1:]
    Wm1_d = W_m1[:L]
    Wm1_s = W_m1[L:2 * L]
    wm1_e = W_m1[2 * L]
    Wu_e = W_u[:L]
    Wu_a = W_u[L:]
    Wmst_e = W_mst[:L]
    Wmst_h = W_mst[L:]
    Wp1_s = W_p1[:L]
    Wp1_d = W_p1[L:2 * L]
    wp1_e = W_p1[2 * L]

    h = jnp.zeros((n, L), jnp.float32)
    prev_tree = x[:, 0]

    def dense_tail(encoded, aggr, h):
        u = _leaky(encoded @ Wu_e + aggr @ Wu_a)
        gi = u @ W_ih.T
        gh = h @ W_hh.T
        i_r, i_z, i_n = jnp.split(gi, 3, axis=1)
        h_r, h_z, h_n = jnp.split(gh, 3, axis=1)
        r = jax.nn.sigmoid(i_r + h_r)
        z = jax.nn.sigmoid(i_z + h_z)
        ng = jnp.tanh(i_n + r * h_n)
        return (1.0 - z) * ng + z * h

    def select(prev_tree, mst_logits):
        nt = jnp.where(prev_tree.astype(bool)[:, None], -1e9, mst_logits)
        nt = nt.reshape(G, -1)
        chosen = jnp.argmax(nt, axis=-1)
        pt = prev_tree.reshape(G, -1)
        pt = pt.at[jnp.arange(G), chosen].set(1.0)
        return pt.reshape(-1)

    for step in range(steps):
        encoded = jax.nn.relu(prev_tree[:, None] * W_enc0[None, :] + h @ W_enc1)
        if step == 0:
            g = _leaky(_leaky(wm1_e) @ W_m2)
            ea_max = jax.ops.segment_max(ea, dst, num_segments=n)
            ea_min = jax.ops.segment_min(ea, dst, num_segments=n)
            has_edge = jnp.isfinite(ea_max)
            ea_max = jnp.where(has_edge, ea_max, 0.0)
            ea_min = jnp.where(has_edge, ea_min, 0.0)
            aggr = jnp.where(g[None, :] > 0, ea_max[:, None] * g[None, :],
                             ea_min[:, None] * g[None, :])
        else:
            A = encoded @ Wm1_d
            B = encoded @ Wm1_s
            pre = A[dst] + B[src] + ea[:, None] * wm1_e[None, :]
            m = _leaky(_leaky(pre) @ W_m2)
            aggr = jax.ops.segment_max(m, dst, num_segments=n)
            aggr = jnp.where(jnp.isfinite(aggr), aggr, 0.0)
        h = dense_tail(encoded, aggr, h)
        if step < steps - 1:
            mst_logits = encoded @ Wmst_e + h @ Wmst_h
            prev_tree = select(prev_tree, mst_logits)

    C = h @ Wp1_s
    D = h @ Wp1_d
    p_in = C[src] + D[dst] + ea[:, None] * wp1_e[None, :]
    p_out = (jax.nn.relu(p_in) @ W_p2)[:, 0]
    pred_logits = jnp.full((n, n), -1e9, jnp.float32).at[src, dst].set(p_out)
    return pred_logits


# speedup vs baseline: 2.2636x; 1.8894x over previous
import functools

import jax
import jax.numpy as jnp
from jax import lax
from jax.experimental import pallas as pl
from jax.experimental.pallas import tpu as pltpu
from jax.experimental.pallas import tpu_sc as plsc

G = 16
N = 4096
E = 131072
L = 128

NBLK = 8
BN = N // NBLK
EBLK = 128
BE = E // EBLK

_NEG = -1e9


def _leaky(v):
    return jnp.where(v >= 0, v, 0.01 * v)


_NC, _NS = 2, 16
_NW = _NC * _NS
_EPW = E // _NW
_GCH = 512
_NCH = _EPW // _GCH


def _sc_gather2_body(a_hbm, b_hbm, dst_hbm, src_hbm, adst_hbm, bsrc_hbm,
                     idx_v, rows_v, sem):
    wid = lax.axis_index("s") * _NC + lax.axis_index("c")

    def chunk(i, _):
        base = wid * _EPW + i * _GCH
        pltpu.sync_copy(dst_hbm.at[pl.ds(base, _GCH)], idx_v)
        pltpu.async_copy(a_hbm.at[idx_v], rows_v, sem).wait()
        pltpu.sync_copy(rows_v, adst_hbm.at[pl.ds(base, _GCH)])
        pltpu.sync_copy(src_hbm.at[pl.ds(base, _GCH)], idx_v)
        pltpu.async_copy(b_hbm.at[idx_v], rows_v, sem).wait()
        pltpu.sync_copy(rows_v, bsrc_hbm.at[pl.ds(base, _GCH)])
        return ()

    lax.fori_loop(0, _NCH, chunk, (), unroll=False)


def _sc_gather2(a, b, dst, src):
    mesh = plsc.VectorSubcoreMesh(core_axis_name="c", subcore_axis_name="s")
    f = pl.kernel(
        _sc_gather2_body,
        mesh=mesh,
        out_type=(
            jax.ShapeDtypeStruct((E, L), jnp.float32),
            jax.ShapeDtypeStruct((E, L), jnp.float32),
        ),
        scratch_types=[
            pltpu.VMEM((_GCH,), jnp.int32),
            pltpu.VMEM((_GCH, L), jnp.float32),
            pltpu.SemaphoreType.DMA,
        ],
    )
    return f(a, b, dst, src)


def _prep_body(pt_ref, h_ref, w0_ref, wenc1_ref, wm1d_ref, wm1s_ref,
               enc_ref, a_ref, b_ref):
    enc = jax.nn.relu(pt_ref[...] * w0_ref[...] +
                      jnp.dot(h_ref[...], wenc1_ref[...],
                              preferred_element_type=jnp.float32))
    enc_ref[...] = enc
    a_ref[...] = jnp.dot(enc, wm1d_ref[...], preferred_element_type=jnp.float32)
    b_ref[...] = jnp.dot(enc, wm1s_ref[...], preferred_element_type=jnp.float32)


def _prep(pt, h, w0, wenc1, wm1d, wm1s):
    row = pl.BlockSpec((BN, L), lambda i: (i, 0))
    col = pl.BlockSpec((BN, 1), lambda i: (i, 0))
    full = pl.BlockSpec((L, L), lambda i: (0, 0))
    vec = pl.BlockSpec((1, L), lambda i: (0, 0))
    return pl.pallas_call(
        _prep_body,
        grid=(NBLK,),
        in_specs=[col, row, vec, full, full, full],
        out_specs=[row, row, row],
        out_shape=[jax.ShapeDtypeStruct((N, L), jnp.float32)] * 3,
    )(pt, h, w0, wenc1, wm1d, wm1s)


def _mmlp_body(a_ref, b_ref, ea_ref, w_ref, wm2_ref, o_ref):
    pre = a_ref[...] + b_ref[...] + ea_ref[...] * w_ref[...]
    q = _leaky(pre)
    o_ref[...] = _leaky(jnp.dot(q, wm2_ref[...],
                                preferred_element_type=jnp.float32))


def _mmlp(adst, bsrc, ea2, w, wm2):
    row = pl.BlockSpec((BE, L), lambda i: (i, 0))
    col = pl.BlockSpec((BE, 1), lambda i: (i, 0))
    full = pl.BlockSpec((L, L), lambda i: (0, 0))
    vec = pl.BlockSpec((1, L), lambda i: (0, 0))
    return pl.pallas_call(
        _mmlp_body,
        grid=(EBLK,),
        in_specs=[row, row, col, vec, full],
        out_specs=row,
        out_shape=jax.ShapeDtypeStruct((E, L), jnp.float32),
    )(adst, bsrc, ea2, w, wm2)


def _gru(u, gh_ref, h, w_iht_ref):
    gi = jnp.dot(u, w_iht_ref[...], preferred_element_type=jnp.float32)
    gh = gh_ref
    r = jax.nn.sigmoid(gi[:, :L] + gh[:, :L])
    z = jax.nn.sigmoid(gi[:, L:2 * L] + gh[:, L:2 * L])
    ng = jnp.tanh(gi[:, 2 * L:] + r * gh[:, 2 * L:])
    return (1.0 - z) * ng + z * h


def _update_body(enc_ref, raw_ref, h_ref, wue_ref, wua_ref, wiht_ref,
                 whht_ref, wmste_ref, wmsth_ref, hn_ref, mst_ref):
    raw = raw_ref[...]
    aggr = jnp.where(raw > -jnp.inf, raw, 0.0)
    enc = enc_ref[...]
    h = h_ref[...]
    u = _leaky(jnp.dot(enc, wue_ref[...], preferred_element_type=jnp.float32) +
               jnp.dot(aggr, wua_ref[...], preferred_element_type=jnp.float32))
    gh = jnp.dot(h, whht_ref[...], preferred_element_type=jnp.float32)
    hn = _gru(u, gh, h, wiht_ref)
    hn_ref[...] = hn
    mst_ref[...] = (jnp.dot(enc, wmste_ref[...], preferred_element_type=jnp.float32) +
                    jnp.dot(hn, wmsth_ref[...], preferred_element_type=jnp.float32))


def _update(enc, raw, h, wue, wua, wiht, whht, wmste, wmsth):
    row = pl.BlockSpec((BN, L), lambda i: (i, 0))
    full = pl.BlockSpec((L, L), lambda i: (0, 0))
    full3 = pl.BlockSpec((L, 3 * L), lambda i: (0, 0))
    cvec = pl.BlockSpec((L, 1), lambda i: (0, 0))
    col = pl.BlockSpec((BN, 1), lambda i: (i, 0))
    return pl.pallas_call(
        _update_body,
        grid=(NBLK,),
        in_specs=[row, row, row, full, full, full3, full3, cvec, cvec],
        out_specs=[row, col],
        out_shape=[jax.ShapeDtypeStruct((N, L), jnp.float32),
                   jax.ShapeDtypeStruct((N, 1), jnp.float32)],
    )(enc, raw, h, wue, wua, wiht, whht, wmste, wmsth)


def _update0_body(emax_ref, emin_ref, wm1e_ref, wm2_ref, wua_ref, wiht_ref,
                  wmsth_ref, hn_ref, mst_ref):
    g = _leaky(jnp.dot(_leaky(wm1e_ref[...]), wm2_ref[...],
                       preferred_element_type=jnp.float32))
    emax_raw = emax_ref[...]
    emin_raw = emin_ref[...]
    emax = jnp.where(emax_raw > -jnp.inf, emax_raw, 0.0)
    emin = jnp.where(emin_raw < jnp.inf, emin_raw, 0.0)
    aggr = jnp.where(g > 0, emax * g, emin * g)
    u = _leaky(jnp.dot(aggr, wua_ref[...], preferred_element_type=jnp.float32))
    gi = jnp.dot(u, wiht_ref[...], preferred_element_type=jnp.float32)
    z = jax.nn.sigmoid(gi[:, L:2 * L])
    ng = jnp.tanh(gi[:, 2 * L:])
    hn = (1.0 - z) * ng
    hn_ref[...] = hn
    mst_ref[...] = jnp.dot(hn, wmsth_ref[...], preferred_element_type=jnp.float32)


def _update0(emax, emin, wm1e, wm2, wua, wiht, wmsth):
    row = pl.BlockSpec((BN, L), lambda i: (i, 0))
    col = pl.BlockSpec((BN, 1), lambda i: (i, 0))
    full = pl.BlockSpec((L, L), lambda i: (0, 0))
    full3 = pl.BlockSpec((L, 3 * L), lambda i: (0, 0))
    vec = pl.BlockSpec((1, L), lambda i: (0, 0))
    cvec = pl.BlockSpec((L, 1), lambda i: (0, 0))
    return pl.pallas_call(
        _update0_body,
        grid=(NBLK,),
        in_specs=[col, col, vec, full, full, full3, cvec],
        out_specs=[row, col],
        out_shape=[jax.ShapeDtypeStruct((N, L), jnp.float32),
                   jax.ShapeDtypeStruct((N, 1), jnp.float32)],
    )(emax, emin, wm1e, wm2, wua, wiht, wmsth)


def _select_body(mst_ref, pt_ref, out_ref):
    mst = mst_ref[...]
    pt = pt_ref[...]
    nt = jnp.where(pt != 0, _NEG, mst)
    iota = lax.broadcasted_iota(jnp.int32, (G, N // G), 1)
    rowmax = jnp.max(nt, axis=1, keepdims=True)
    cand = jnp.where(nt == rowmax, iota, jnp.int32(2**30))
    chosen = jnp.min(cand, axis=1, keepdims=True)
    out_ref[...] = jnp.where(iota == chosen, 1.0, pt)


def _select(mst_g, pt_g):
    blk = pl.BlockSpec((G, N // G), lambda: (0, 0))
    return pl.pallas_call(
        _select_body,
        in_specs=[blk, blk],
        out_specs=blk,
        out_shape=jax.ShapeDtypeStruct((G, N // G), jnp.float32),
    )(mst_g, pt_g)


def _final_body(enc_ref, raw_ref, h_ref, wue_ref, wua_ref, wiht_ref,
                whht_ref, wp1s_ref, wp1d_ref, c_ref, d_ref):
    raw = raw_ref[...]
    aggr = jnp.where(raw > -jnp.inf, raw, 0.0)
    enc = enc_ref[...]
    h = h_ref[...]
    u = _leaky(jnp.dot(enc, wue_ref[...], preferred_element_type=jnp.float32) +
               jnp.dot(aggr, wua_ref[...], preferred_element_type=jnp.float32))
    gh = jnp.dot(h, whht_ref[...], preferred_element_type=jnp.float32)
    hn = _gru(u, gh, h, wiht_ref)
    c_ref[...] = jnp.dot(hn, wp1s_ref[...], preferred_element_type=jnp.float32)
    d_ref[...] = jnp.dot(hn, wp1d_ref[...], preferred_element_type=jnp.float32)


def _final_update(enc, raw, h, wue, wua, wiht, whht, wp1s, wp1d):
    row = pl.BlockSpec((BN, L), lambda i: (i, 0))
    full = pl.BlockSpec((L, L), lambda i: (0, 0))
    full3 = pl.BlockSpec((L, 3 * L), lambda i: (0, 0))
    return pl.pallas_call(
        _final_body,
        grid=(NBLK,),
        in_specs=[row, row, row, full, full, full3, full3, full, full],
        out_specs=[row, row],
        out_shape=[jax.ShapeDtypeStruct((N, L), jnp.float32)] * 2,
    )(enc, raw, h, wue, wua, wiht, whht, wp1s, wp1d)


def _pout_body(c_ref, d_ref, ea_ref, w_ref, wp2_ref, o_ref):
    pin = c_ref[...] + d_ref[...] + ea_ref[...] * w_ref[...]
    o_ref[...] = jnp.dot(jax.nn.relu(pin), wp2_ref[...],
                         preferred_element_type=jnp.float32)


def _pout(csrc, ddst, ea2, wp1e, wp2):
    row = pl.BlockSpec((BE, L), lambda i: (i, 0))
    col = pl.BlockSpec((BE, 1), lambda i: (i, 0))
    vec = pl.BlockSpec((1, L), lambda i: (0, 0))
    cvec = pl.BlockSpec((L, 1), lambda i: (0, 0))
    return pl.pallas_call(
        _pout_body,
        grid=(EBLK,),
        in_specs=[row, row, col, vec, cvec],
        out_specs=col,
        out_shape=jax.ShapeDtypeStruct((E, 1), jnp.float32),
    )(csrc, ddst, ea2, wp1e, wp2)


def kernel(x, edge_attr, edge_index, W_enc, W_m1, W_m2, W_u, W_ih, W_hh, W_mst, W_p1, W_p2):
    n = x.shape[0]
    steps = x.shape[1]
    src = edge_index[0]
    dst = edge_index[1]
    ea = edge_attr
    ea2 = ea[:, None]

    w_enc0 = W_enc[0:1]
    W_enc1 = W_enc[1:]
    Wm1_d = W_m1[:L]
    Wm1_s = W_m1[L:2 * L]
    wm1_e = W_m1[2 * L:2 * L + 1]
    Wu_e = W_u[:L]
    Wu_a = W_u[L:]
    W_ihT = W_ih.T
    W_hhT = W_hh.T
    Wmst_e = W_mst[:L]
    Wmst_h = W_mst[L:]
    Wp1_s = W_p1[:L]
    Wp1_d = W_p1[L:2 * L]
    wp1_e = W_p1[2 * L:2 * L + 1]

    pt = x[:, 0]

    ea_max = jax.ops.segment_max(ea, dst, num_segments=n)[:, None]
    ea_min = jax.ops.segment_min(ea, dst, num_segments=n)[:, None]
    h, mst = _update0(ea_max, ea_min, wm1_e, W_m2, Wu_a, W_ihT, Wmst_h)
    pt = _select(mst.reshape(G, n // G), pt.reshape(G, n // G)).reshape(-1)

    for step in range(1, steps):
        enc, A, B = _prep(pt[:, None], h, w_enc0, W_enc1, Wm1_d, Wm1_s)
        adst, bsrc = _sc_gather2(A, B, dst, src)
        m = _mmlp(adst, bsrc, ea2, wm1_e, W_m2)
        raw = jax.ops.segment_max(m, dst, num_segments=n)
        if step < steps - 1:
            h, mst = _update(enc, raw, h, Wu_e, Wu_a, W_ihT, W_hhT,
                             Wmst_e, Wmst_h)
            pt = _select(mst.reshape(G, n // G),
                         pt.reshape(G, n // G)).reshape(-1)
        else:
            C, D = _final_update(enc, raw, h, Wu_e, Wu_a, W_ihT, W_hhT,
                                 Wp1_s, Wp1_d)

    csrc, ddst = _sc_gather2(C, D, src, dst)
    p_out = _pout(csrc, ddst, ea2, wp1_e, W_p2)[:, 0]
    pred_logits = jnp.full((n, n), _NEG, jnp.float32).at[src, dst].set(p_out)
    return pred_logits
